# baseline (speedup 1.0000x reference)
import numpy as np

import jax
import jax.numpy as jnp
from jax.experimental import pallas as pl


def _take_mask(C, H, W):
    mk = jax.random.key(42)
    kc, ks = jax.random.split(mk)
    cm = jax.random.randint(kc, (C,), 0, 2).astype(jnp.uint8).astype(bool)
    spatial = jax.random.randint(ks, (H,), 0, 2)
    neg_idx = jnp.bitwise_not(spatial) % W
    pos_idx = spatial % W
    neg_hit = jnp.zeros((W,), dtype=bool).at[neg_idx].set(True)
    pos_hit = jnp.zeros((W,), dtype=bool).at[pos_idx].set(True)
    return pos_hit[None, :] | (~neg_hit[None, :] & cm[:, None])


def _select4d(lst, gui):
    N, C, H, W = lst.shape
    mask = _take_mask(C, H, W).astype(jnp.float32).reshape(C, 1, W)

    def body(m_ref, a_ref, b_ref, o1_ref, o2_ref):
        m = (m_ref[...] != 0.0)[None]
        a = a_ref[...]
        b = b_ref[...]
        o1_ref[...] = jnp.where(m, b, a)
        o2_ref[...] = jnp.where(m, a, b)

    CB = 32
    while C % CB:
        CB //= 2
    grid = (N, C // CB)
    data_spec = pl.BlockSpec((1, CB, H, W), lambda n, c: (n, c, 0, 0))
    mask_spec = pl.BlockSpec((CB, 1, W), lambda n, c: (c, 0, 0))
    return tuple(pl.pallas_call(
        body,
        grid=grid,
        in_specs=[mask_spec, data_spec, data_spec],
        out_specs=[data_spec, data_spec],
        out_shape=[
            jax.ShapeDtypeStruct(lst.shape, lst.dtype),
            jax.ShapeDtypeStruct(gui.shape, gui.dtype),
        ],
    )(mask, lst, gui))


def _select_flat128(lst, gui):
    N, C, H, W = lst.shape
    F = H * W
    period = int(np.lcm(W, 128))
    P = period // 128
    R = F // period
    take = _take_mask(C, H, W)
    idx = jnp.arange(period) % W
    mask = take[:, idx].astype(jnp.float32).reshape(C, P, 128)

    lstf = lst.reshape(N, C, F // 128, 128)
    guif = gui.reshape(N, C, F // 128, 128)

    def body(m_ref, a_ref, b_ref, o1_ref, o2_ref):
        m = (m_ref[...] != 0.0)[None]
        for r in range(R):
            sl = (slice(None), slice(None), pl.ds(r * P, P), slice(None))
            a = a_ref[sl]
            b = b_ref[sl]
            o1_ref[sl] = jnp.where(m, b, a)
            o2_ref[sl] = jnp.where(m, a, b)

    CB = 16
    while C % CB:
        CB //= 2
    grid = (N, C // CB)
    data_spec = pl.BlockSpec((1, CB, F // 128, 128), lambda n, c: (n, c, 0, 0))
    mask_spec = pl.BlockSpec((CB, P, 128), lambda n, c: (c, 0, 0))
    o1, o2 = pl.pallas_call(
        body,
        grid=grid,
        in_specs=[mask_spec, data_spec, data_spec],
        out_specs=[data_spec, data_spec],
        out_shape=[
            jax.ShapeDtypeStruct(lstf.shape, lst.dtype),
            jax.ShapeDtypeStruct(guif.shape, gui.dtype),
        ],
    )(mask, lstf, guif)
    return o1.reshape(N, C, H, W), o2.reshape(N, C, H, W)


def kernel(lst, gui):
    N, C, H, W = lst.shape
    F = H * W
    if F % 128 == 0 and F % int(np.lcm(W, 128)) == 0:
        return _select_flat128(lst, gui)
    return _select4d(lst, gui)


# speedup vs baseline: 4.1261x; 4.1261x over previous
import numpy as np

import jax
import jax.numpy as jnp
from jax.experimental import pallas as pl


def _take_mask(C, H, W):
    mk = jax.random.key(42)
    kc, ks = jax.random.split(mk)
    cm = jax.random.randint(kc, (C,), 0, 2).astype(jnp.uint8).astype(bool)
    spatial = jax.random.randint(ks, (H,), 0, 2)
    neg_idx = jnp.bitwise_not(spatial) % W
    pos_idx = spatial % W
    neg_hit = jnp.zeros((W,), dtype=bool).at[neg_idx].set(True)
    pos_hit = jnp.zeros((W,), dtype=bool).at[pos_idx].set(True)
    return pos_hit[None, :] | (~neg_hit[None, :] & cm[:, None])


def _select4d(lst, gui):
    N, C, H, W = lst.shape
    mask = _take_mask(C, H, W).astype(jnp.float32).reshape(C, 1, W)

    def body(m_ref, a_ref, b_ref, o1_ref, o2_ref):
        m = (m_ref[...] != 0.0)[None]
        a = a_ref[...]
        b = b_ref[...]
        o1_ref[...] = jnp.where(m, b, a)
        o2_ref[...] = jnp.where(m, a, b)

    CB = 32
    while C % CB:
        CB //= 2
    grid = (N, C // CB)
    data_spec = pl.BlockSpec((1, CB, H, W), lambda n, c: (n, c, 0, 0))
    mask_spec = pl.BlockSpec((CB, 1, W), lambda n, c: (c, 0, 0))
    return tuple(pl.pallas_call(
        body,
        grid=grid,
        in_specs=[mask_spec, data_spec, data_spec],
        out_specs=[data_spec, data_spec],
        out_shape=[
            jax.ShapeDtypeStruct(lst.shape, lst.dtype),
            jax.ShapeDtypeStruct(gui.shape, gui.dtype),
        ],
    )(mask, lst, gui))


def _select_flat128(lst, gui):
    N, C, H, W = lst.shape
    F = H * W
    period = int(np.lcm(W, 128))
    P = period // 128
    R = F // period
    take = _take_mask(C, H, W)
    idx = jnp.arange(period) % W
    mask = take[:, idx].astype(jnp.float32).reshape(C, P, 128)

    lstf = lst.reshape(N, C, F // 128, 128)
    guif = gui.reshape(N, C, F // 128, 128)

    def body(m_ref, a_ref, b_ref, o1_ref, o2_ref):
        m = (m_ref[...] != 0.0)[None]
        for r in range(R):
            sl = (slice(None), slice(None), pl.ds(r * P, P), slice(None))
            a = a_ref[sl]
            b = b_ref[sl]
            o1_ref[sl] = jnp.where(m, b, a)
            o2_ref[sl] = jnp.where(m, a, b)

    CB = 16
    while C % CB:
        CB //= 2
    grid = (N, C // CB)
    data_spec = pl.BlockSpec((1, CB, F // 128, 128), lambda n, c: (n, c, 0, 0))
    mask_spec = pl.BlockSpec((CB, P, 128), lambda n, c: (c, 0, 0))
    o1, o2 = pl.pallas_call(
        body,
        grid=grid,
        in_specs=[mask_spec, data_spec, data_spec],
        out_specs=[data_spec, data_spec],
        out_shape=[
            jax.ShapeDtypeStruct(lstf.shape, lst.dtype),
            jax.ShapeDtypeStruct(guif.shape, gui.dtype),
        ],
    )(mask, lstf, guif)
    return o1.reshape(N, C, H, W), o2.reshape(N, C, H, W)


def kernel(lst, gui):
    return _select4d(lst, gui)
